# baseline (speedup 1.0000x reference)
import functools

import jax
import jax.numpy as jnp
from jax import lax
from jax.experimental import pallas as pl
from jax.experimental.pallas import tpu as pltpu
from jax.experimental.pallas import tpu_sc as plsc

_K = 8
_BANDWIDTH = 0.05
_TEMP = 2.0 * _BANDWIDTH * _BANDWIDTH
_EPS = 1e-08

_QB = 512
_NB = 2048
_GROUPS = 8

_NC = 2
_NS = 16
_NW = _NC * _NS
_CH = 128


def _topk_body(c_ref, pt_ref, w_ref, idx_ref, wt_ref, tv_ref, ti_ref, d_ref):
    nn = pl.num_programs(1)
    ni = pl.program_id(1)
    qb = c_ref.shape[0]
    nb = pt_ref.shape[1]

    @pl.when(ni == 0)
    def _init():
        tv_ref[:, 0:_K] = jnp.full((qb, _K), jnp.inf, jnp.float32)
        ti_ref[:, 0:_K] = jnp.zeros((qb, _K), jnp.int32)

    c = c_ref[...]
    pt = pt_ref[...]
    w = w_ref[...]
    csq = jnp.sum(c * c, axis=1, keepdims=True)
    psq = jnp.sum(pt * pt, axis=0, keepdims=True)
    d = csq + psq - 2.0 * jnp.dot(c, pt, preferred_element_type=jnp.float32)
    d = jnp.maximum(d, 0.0)
    d = d * (1.0 / (w * w + _EPS))

    t = tv_ref[:, _K - 1:_K]
    hits = d < t
    cnt = jnp.sum(hits.astype(jnp.int32), axis=1, keepdims=True)
    tv_ref[:, _K:2 * _K] = jnp.full((qb, _K), jnp.inf, jnp.float32)

    gr = qb // _GROUPS
    col = lax.broadcasted_iota(jnp.int32, (gr, nb), 1)
    col2 = lax.broadcasted_iota(jnp.int32, (gr, 2 * _K), 1)
    base = ni * nb
    big = jnp.int32(2**30)
    for g in range(_GROUPS):
        r0 = g * gr
        cmax = jnp.max(cnt[r0:r0 + gr, :])

        @pl.when(cmax > 0)
        def _stage(g=g, r0=r0):
            d_ref[r0:r0 + gr, :] = d[r0:r0 + gr, :]

        for k in range(_K):
            @pl.when(cmax > k)
            def _extract(k=k, r0=r0):
                dd = d_ref[r0:r0 + gr, :]
                m = jnp.min(dd, axis=1, keepdims=True)
                cand = jnp.where(dd == m, col, big)
                am = jnp.min(cand, axis=1, keepdims=True)
                tv_ref[r0:r0 + gr, _K + k:_K + k + 1] = m
                ti_ref[r0:r0 + gr, _K + k:_K + k + 1] = am + base
                d_ref[r0:r0 + gr, :] = jnp.where(cand == am, jnp.inf, dd)

        @pl.when(cmax > 0)
        def _merge(r0=r0):
            vals = tv_ref[r0:r0 + gr, :]
            idxs = ti_ref[r0:r0 + gr, :]
            newv = []
            newi = []
            for k in range(_K):
                m = jnp.min(vals, axis=1, keepdims=True)
                cand = jnp.where(vals == m, col2, big)
                am = jnp.min(cand, axis=1, keepdims=True)
                pick = cand == am
                ik = jnp.sum(jnp.where(pick, idxs, 0), axis=1, keepdims=True)
                newv.append(m)
                newi.append(ik)
                vals = jnp.where(pick, jnp.inf, vals)
            tv_ref[r0:r0 + gr, 0:_K] = jnp.concatenate(newv, axis=1)
            ti_ref[r0:r0 + gr, 0:_K] = jnp.concatenate(newi, axis=1)

    @pl.when(ni == nn - 1)
    def _emit():
        v8 = tv_ref[:, 0:_K]
        i8 = ti_ref[:, 0:_K]
        e = jnp.exp(-(v8 - v8[:, 0:1]) / _TEMP)
        wt_ref[...] = e / jnp.sum(e, axis=1, keepdims=True)
        idx_ref[...] = i8


def _topk_call(coords, pt, w2, np_pad):
    b = coords.shape[0]
    dm = coords.shape[1]
    qb = min(_QB, b)
    nq = b // qb
    nn = np_pad // _NB
    return pl.pallas_call(
        _topk_body,
        grid=(nq, nn),
        in_specs=[
            pl.BlockSpec((qb, dm), lambda qi, ni: (qi, 0)),
            pl.BlockSpec((dm, _NB), lambda qi, ni: (0, ni)),
            pl.BlockSpec((1, _NB), lambda qi, ni: (0, ni)),
        ],
        out_specs=[
            pl.BlockSpec((qb, _K), lambda qi, ni: (qi, 0)),
            pl.BlockSpec((qb, _K), lambda qi, ni: (qi, 0)),
        ],
        out_shape=[
            jax.ShapeDtypeStruct((b, _K), jnp.int32),
            jax.ShapeDtypeStruct((b, _K), jnp.float32),
        ],
        scratch_shapes=[
            pltpu.VMEM((qb, 2 * _K), jnp.float32),
            pltpu.VMEM((qb, 2 * _K), jnp.int32),
            pltpu.VMEM((qb, _NB), jnp.float32),
        ],
        compiler_params=pltpu.CompilerParams(
            dimension_semantics=("arbitrary", "arbitrary")),
    )(coords, pt, w2)


def _gather(features, idx_flat):
    b2 = idx_flat.shape[0]
    f = features.shape[1]
    per_w = b2 // _NW
    nch = per_w // _CH
    mesh = plsc.VectorSubcoreMesh(core_axis_name="c", subcore_axis_name="s")

    @functools.partial(
        pl.kernel,
        mesh=mesh,
        out_type=jax.ShapeDtypeStruct((b2, f), jnp.float32),
        scratch_types=[
            pltpu.VMEM((_CH,), jnp.int32),
            pltpu.VMEM((_CH, f), jnp.float32),
            pltpu.SemaphoreType.DMA,
        ],
    )
    def gather_k(feat_hbm, idx_hbm, out_hbm, idx_v, rows_v, sem):
        wid = lax.axis_index("s") * _NC + lax.axis_index("c")
        base = wid * per_w
        for ch in range(nch):
            off = base + ch * _CH
            pltpu.sync_copy(idx_hbm.at[pl.ds(off, _CH)], idx_v)
            pltpu.async_copy(feat_hbm.at[idx_v], rows_v, sem).wait()
            pltpu.sync_copy(rows_v, out_hbm.at[pl.ds(off, _CH)])

    return gather_k(features, idx_flat)


def _wsum_body(rows_ref, w_ref, out_ref):
    r = rows_ref[...]
    w = w_ref[...]
    out_ref[...] = jnp.sum(r * w[:, :, None], axis=1)


def _wsum_call(rows, wt):
    b, k, f = rows.shape
    qb = min(_QB, b)
    return pl.pallas_call(
        _wsum_body,
        grid=(b // qb,),
        in_specs=[
            pl.BlockSpec((qb, k, f), lambda qi: (qi, 0, 0)),
            pl.BlockSpec((qb, k), lambda qi: (qi, 0)),
        ],
        out_specs=pl.BlockSpec((qb, f), lambda qi: (qi, 0)),
        out_shape=jax.ShapeDtypeStruct((b, f), jnp.float32),
    )(rows, wt)


def kernel(coords, positions, weights, features):
    b, dm = coords.shape
    n = positions.shape[0]
    f = features.shape[1]
    np_pad = ((n + _NB - 1) // _NB) * _NB
    pad = np_pad - n
    pos_p = jnp.pad(positions, ((0, pad), (0, 0)), constant_values=1e6)
    w_p = jnp.pad(weights, (0, pad))
    pt = pos_p.T
    w2 = w_p[None, :]

    idx8, wt8 = _topk_call(coords, pt, w2, np_pad)
    rows = _gather(features, idx8.reshape(-1))
    return _wsum_call(rows.reshape(b, _K, f), wt8)


# speedup vs baseline: 2.5476x; 2.5476x over previous
import functools

import jax
import jax.numpy as jnp
from jax import lax
from jax.experimental import pallas as pl
from jax.experimental.pallas import tpu as pltpu
from jax.experimental.pallas import tpu_sc as plsc

_K = 8
_BANDWIDTH = 0.05
_TEMP = 2.0 * _BANDWIDTH * _BANDWIDTH
_EPS = 1e-08

_QB = 512
_NB = 2048
_GROUPS = 8

_NC = 2
_NS = 16
_NW = _NC * _NS
_CH = 128


def _topk_body(c_ref, pt_ref, w_ref, idx_ref, wt_ref, tv_ref, ti_ref, d_ref):
    nn = pl.num_programs(1)
    ni = pl.program_id(1)
    qb = c_ref.shape[0]
    nb = pt_ref.shape[1]

    @pl.when(ni == 0)
    def _init():
        tv_ref[:, 0:_K] = jnp.full((qb, _K), jnp.inf, jnp.float32)
        ti_ref[:, 0:_K] = jnp.zeros((qb, _K), jnp.int32)

    c = c_ref[...]
    pt = pt_ref[...]
    w = w_ref[...]
    csq = jnp.sum(c * c, axis=1, keepdims=True)
    psq = jnp.sum(pt * pt, axis=0, keepdims=True)
    d = csq + psq - 2.0 * jnp.dot(c, pt, preferred_element_type=jnp.float32)
    d = jnp.maximum(d, 0.0)
    d = d * (1.0 / (w * w + _EPS))

    t = tv_ref[:, _K - 1:_K]
    hits = d < t
    cnt = jnp.sum(hits.astype(jnp.int32), axis=1, keepdims=True)
    tv_ref[:, _K:2 * _K] = jnp.full((qb, _K), jnp.inf, jnp.float32)

    cmax = jnp.max(cnt)
    col = lax.broadcasted_iota(jnp.int32, (qb, nb), 1)
    base = ni * nb
    big = jnp.int32(2**30)

    @pl.when(cmax > 0)
    def _extract0():
        m = jnp.min(d, axis=1, keepdims=True)
        cand = jnp.where(d == m, col, big)
        am = jnp.min(cand, axis=1, keepdims=True)
        tv_ref[:, _K:_K + 1] = m
        ti_ref[:, _K:_K + 1] = am + base

        @pl.when(cmax > 1)
        def _spill():
            d_ref[...] = jnp.where(cand == am, jnp.inf, d)

    for k in range(1, _K):
        @pl.when(cmax > k)
        def _extract(k=k):
            dd = d_ref[...]
            m = jnp.min(dd, axis=1, keepdims=True)
            cand = jnp.where(dd == m, col, big)
            am = jnp.min(cand, axis=1, keepdims=True)
            tv_ref[:, _K + k:_K + k + 1] = m
            ti_ref[:, _K + k:_K + k + 1] = am + base
            if k < _K - 1:
                d_ref[...] = jnp.where(cand == am, jnp.inf, dd)

    @pl.when(cmax > 0)
    def _merge():
        vals = tv_ref[...]
        idxs = ti_ref[...]
        col2 = lax.broadcasted_iota(jnp.int32, (qb, 2 * _K), 1)
        newv = []
        newi = []
        for k in range(_K):
            m = jnp.min(vals, axis=1, keepdims=True)
            cand = jnp.where(vals == m, col2, big)
            am = jnp.min(cand, axis=1, keepdims=True)
            pick = cand == am
            ik = jnp.sum(jnp.where(pick, idxs, 0), axis=1, keepdims=True)
            newv.append(m)
            newi.append(ik)
            vals = jnp.where(pick, jnp.inf, vals)
        tv_ref[:, 0:_K] = jnp.concatenate(newv, axis=1)
        ti_ref[:, 0:_K] = jnp.concatenate(newi, axis=1)

    @pl.when(ni == nn - 1)
    def _emit():
        v8 = tv_ref[:, 0:_K]
        i8 = ti_ref[:, 0:_K]
        e = jnp.exp(-(v8 - v8[:, 0:1]) / _TEMP)
        wt_ref[...] = e / jnp.sum(e, axis=1, keepdims=True)
        idx_ref[...] = i8


def _topk_call(coords, pt, w2, np_pad):
    b = coords.shape[0]
    dm = coords.shape[1]
    qb = min(_QB, b)
    nq = b // qb
    nn = np_pad // _NB
    return pl.pallas_call(
        _topk_body,
        grid=(nq, nn),
        in_specs=[
            pl.BlockSpec((qb, dm), lambda qi, ni: (qi, 0)),
            pl.BlockSpec((dm, _NB), lambda qi, ni: (0, ni)),
            pl.BlockSpec((1, _NB), lambda qi, ni: (0, ni)),
        ],
        out_specs=[
            pl.BlockSpec((qb, _K), lambda qi, ni: (qi, 0)),
            pl.BlockSpec((qb, _K), lambda qi, ni: (qi, 0)),
        ],
        out_shape=[
            jax.ShapeDtypeStruct((b, _K), jnp.int32),
            jax.ShapeDtypeStruct((b, _K), jnp.float32),
        ],
        scratch_shapes=[
            pltpu.VMEM((qb, 2 * _K), jnp.float32),
            pltpu.VMEM((qb, 2 * _K), jnp.int32),
            pltpu.VMEM((qb, _NB), jnp.float32),
        ],
        compiler_params=pltpu.CompilerParams(
            dimension_semantics=("arbitrary", "arbitrary")),
    )(coords, pt, w2)


def _gather(features, idx_flat):
    b2 = idx_flat.shape[0]
    f = features.shape[1]
    per_w = b2 // _NW
    nch = per_w // _CH
    mesh = plsc.VectorSubcoreMesh(core_axis_name="c", subcore_axis_name="s")

    @functools.partial(
        pl.kernel,
        mesh=mesh,
        out_type=jax.ShapeDtypeStruct((b2, f), jnp.float32),
        scratch_types=[
            pltpu.VMEM((_CH,), jnp.int32),
            pltpu.VMEM((_CH, f), jnp.float32),
            pltpu.SemaphoreType.DMA,
        ],
    )
    def gather_k(feat_hbm, idx_hbm, out_hbm, idx_v, rows_v, sem):
        wid = lax.axis_index("s") * _NC + lax.axis_index("c")
        base = wid * per_w
        for ch in range(nch):
            off = base + ch * _CH
            pltpu.sync_copy(idx_hbm.at[pl.ds(off, _CH)], idx_v)
            pltpu.async_copy(feat_hbm.at[idx_v], rows_v, sem).wait()
            pltpu.sync_copy(rows_v, out_hbm.at[pl.ds(off, _CH)])

    return gather_k(features, idx_flat)


def _wsum_body(rows_ref, w_ref, out_ref):
    r = rows_ref[...]
    w = w_ref[...]
    out_ref[...] = jnp.sum(r * w[:, :, None], axis=1)


def _wsum_call(rows, wt):
    b, k, f = rows.shape
    qb = min(_QB, b)
    return pl.pallas_call(
        _wsum_body,
        grid=(b // qb,),
        in_specs=[
            pl.BlockSpec((qb, k, f), lambda qi: (qi, 0, 0)),
            pl.BlockSpec((qb, k), lambda qi: (qi, 0)),
        ],
        out_specs=pl.BlockSpec((qb, f), lambda qi: (qi, 0)),
        out_shape=jax.ShapeDtypeStruct((b, f), jnp.float32),
    )(rows, wt)


def kernel(coords, positions, weights, features):
    b, dm = coords.shape
    n = positions.shape[0]
    f = features.shape[1]
    np_pad = ((n + _NB - 1) // _NB) * _NB
    pad = np_pad - n
    pos_p = jnp.pad(positions, ((0, pad), (0, 0)), constant_values=1e6)
    w_p = jnp.pad(weights, (0, pad))
    pt = pos_p.T
    w2 = w_p[None, :]

    idx8, wt8 = _topk_call(coords, pt, w2, np_pad)
    rows = _gather(features, idx8.reshape(-1))
    return _wsum_call(rows.reshape(b, _K, f), wt8)


# speedup vs baseline: 3.2626x; 1.2807x over previous
import functools

import jax
import jax.numpy as jnp
from jax import lax
from jax.experimental import pallas as pl
from jax.experimental.pallas import tpu as pltpu
from jax.experimental.pallas import tpu_sc as plsc

_K = 8
_BANDWIDTH = 0.05
_TEMP = 2.0 * _BANDWIDTH * _BANDWIDTH
_EPS = 1e-08

_QB = 512
_NB = 2048
_GROUPS = 8

_NC = 2
_NS = 16
_NW = _NC * _NS
_CH = 128


def _topk_body(c_ref, pt_ref, w_ref, idx_ref, wt_ref, tv_ref, ti_ref, d_ref):
    nn = pl.num_programs(1)
    ni = pl.program_id(1)
    qb = c_ref.shape[0]
    nb = pt_ref.shape[1]

    @pl.when(ni == 0)
    def _init():
        tv_ref[:, 0:_K] = jnp.full((qb, _K), jnp.inf, jnp.float32)
        ti_ref[:, 0:_K] = jnp.zeros((qb, _K), jnp.float32)

    c = c_ref[...]
    pt = pt_ref[...]
    w = w_ref[...]
    csq = jnp.sum(c * c, axis=1, keepdims=True)
    psq = jnp.sum(pt * pt, axis=0, keepdims=True)
    d = csq + psq - 2.0 * jnp.dot(c, pt, preferred_element_type=jnp.float32)
    d = jnp.maximum(d, 0.0)
    d = d * (1.0 / (w * w + _EPS))

    t = tv_ref[:, _K - 1:_K]
    hits = d < t
    cnt = jnp.sum(jnp.where(hits, 1.0, 0.0), axis=1, keepdims=True)
    tv_ref[:, _K:2 * _K] = jnp.full((qb, _K), jnp.inf, jnp.float32)

    cmax = jnp.max(cnt)
    col = lax.broadcasted_iota(jnp.int32, (qb, nb), 1).astype(jnp.float32)
    base = jnp.float32(ni * nb)
    big = jnp.float32(1e9)

    @pl.when(cmax > 0)
    def _extract0():
        m = jnp.min(d, axis=1, keepdims=True)
        cand = jnp.where(d == m, col, big)
        am = jnp.min(cand, axis=1, keepdims=True)
        tv_ref[:, _K:_K + 1] = m
        ti_ref[:, _K:_K + 1] = am + base

        @pl.when(cmax > 1)
        def _spill():
            d_ref[...] = jnp.where(cand == am, jnp.inf, d)

    for k in range(1, _K):
        @pl.when(cmax > k)
        def _extract(k=k):
            dd = d_ref[...]
            m = jnp.min(dd, axis=1, keepdims=True)
            cand = jnp.where(dd == m, col, big)
            am = jnp.min(cand, axis=1, keepdims=True)
            tv_ref[:, _K + k:_K + k + 1] = m
            ti_ref[:, _K + k:_K + k + 1] = am + base
            if k < _K - 1:
                d_ref[...] = jnp.where(cand == am, jnp.inf, dd)

    @pl.when(cmax > 0)
    def _merge():
        vals = tv_ref[...]
        idxs = ti_ref[...]
        col2 = lax.broadcasted_iota(jnp.int32, (qb, 2 * _K), 1).astype(jnp.float32)
        newv = []
        newi = []
        for k in range(_K):
            m = jnp.min(vals, axis=1, keepdims=True)
            cand = jnp.where(vals == m, col2, big)
            am = jnp.min(cand, axis=1, keepdims=True)
            pick = cand == am
            ik = jnp.sum(jnp.where(pick, idxs, 0.0), axis=1, keepdims=True)
            newv.append(m)
            newi.append(ik)
            vals = jnp.where(pick, jnp.inf, vals)
        tv_ref[:, 0:_K] = jnp.concatenate(newv, axis=1)
        ti_ref[:, 0:_K] = jnp.concatenate(newi, axis=1)

    @pl.when(ni == nn - 1)
    def _emit():
        v8 = tv_ref[:, 0:_K]
        i8 = ti_ref[:, 0:_K]
        e = jnp.exp(-(v8 - v8[:, 0:1]) / _TEMP)
        wt_ref[...] = e / jnp.sum(e, axis=1, keepdims=True)
        idx_ref[...] = i8.astype(jnp.int32)


def _topk_call(coords, pt, w2, np_pad):
    b = coords.shape[0]
    dm = coords.shape[1]
    qb = min(_QB, b)
    nq = b // qb
    nn = np_pad // _NB
    return pl.pallas_call(
        _topk_body,
        grid=(nq, nn),
        in_specs=[
            pl.BlockSpec((qb, dm), lambda qi, ni: (qi, 0)),
            pl.BlockSpec((dm, _NB), lambda qi, ni: (0, ni)),
            pl.BlockSpec((1, _NB), lambda qi, ni: (0, ni)),
        ],
        out_specs=[
            pl.BlockSpec((qb, _K), lambda qi, ni: (qi, 0)),
            pl.BlockSpec((qb, _K), lambda qi, ni: (qi, 0)),
        ],
        out_shape=[
            jax.ShapeDtypeStruct((b, _K), jnp.int32),
            jax.ShapeDtypeStruct((b, _K), jnp.float32),
        ],
        scratch_shapes=[
            pltpu.VMEM((qb, 2 * _K), jnp.float32),
            pltpu.VMEM((qb, 2 * _K), jnp.float32),
            pltpu.VMEM((qb, _NB), jnp.float32),
        ],
        compiler_params=pltpu.CompilerParams(
            dimension_semantics=("arbitrary", "arbitrary")),
    )(coords, pt, w2)


def _gather(features, idx_flat):
    b2 = idx_flat.shape[0]
    f = features.shape[1]
    per_w = b2 // _NW
    nch = per_w // _CH
    mesh = plsc.VectorSubcoreMesh(core_axis_name="c", subcore_axis_name="s")

    @functools.partial(
        pl.kernel,
        mesh=mesh,
        out_type=jax.ShapeDtypeStruct((b2, f), jnp.float32),
        scratch_types=[
            pltpu.VMEM((_CH,), jnp.int32),
            pltpu.VMEM((_CH, f), jnp.float32),
            pltpu.SemaphoreType.DMA,
        ],
    )
    def gather_k(feat_hbm, idx_hbm, out_hbm, idx_v, rows_v, sem):
        wid = lax.axis_index("s") * _NC + lax.axis_index("c")
        base = wid * per_w
        for ch in range(nch):
            off = base + ch * _CH
            pltpu.sync_copy(idx_hbm.at[pl.ds(off, _CH)], idx_v)
            pltpu.async_copy(feat_hbm.at[idx_v], rows_v, sem).wait()
            pltpu.sync_copy(rows_v, out_hbm.at[pl.ds(off, _CH)])

    return gather_k(features, idx_flat)


def _wsum_body(rows_ref, w_ref, out_ref):
    r = rows_ref[...]
    w = w_ref[...]
    out_ref[...] = jnp.sum(r * w[:, :, None], axis=1)


def _wsum_call(rows, wt):
    b, k, f = rows.shape
    qb = min(_QB, b)
    return pl.pallas_call(
        _wsum_body,
        grid=(b // qb,),
        in_specs=[
            pl.BlockSpec((qb, k, f), lambda qi: (qi, 0, 0)),
            pl.BlockSpec((qb, k), lambda qi: (qi, 0)),
        ],
        out_specs=pl.BlockSpec((qb, f), lambda qi: (qi, 0)),
        out_shape=jax.ShapeDtypeStruct((b, f), jnp.float32),
    )(rows, wt)


def kernel(coords, positions, weights, features):
    b, dm = coords.shape
    n = positions.shape[0]
    f = features.shape[1]
    np_pad = ((n + _NB - 1) // _NB) * _NB
    pad = np_pad - n
    pos_p = jnp.pad(positions, ((0, pad), (0, 0)), constant_values=1e6)
    w_p = jnp.pad(weights, (0, pad))
    pt = pos_p.T
    w2 = w_p[None, :]

    idx8, wt8 = _topk_call(coords, pt, w2, np_pad)
    rows = _gather(features, idx8.reshape(-1))
    return _wsum_call(rows.reshape(b, _K, f), wt8)
